# baseline (speedup 1.0000x reference)
import functools

import jax
import jax.numpy as jnp
from jax import lax
from jax.experimental import pallas as pl
from jax.experimental.pallas import tpu as pltpu
from jax.experimental.pallas import tpu_sc as plsc

_L = 4096
_D = 1024
_NC = 2
_ROWS_PER_C = _L // _NC
_CHUNK = 256
_NCHUNK = _ROWS_PER_C // _CHUNK
_NBUF = 4
_LAG = 2


def _make_copy_kernel():
    mesh = plsc.ScalarSubcoreMesh(axis_name="c", num_cores=_NC)

    @functools.partial(
        pl.kernel,
        mesh=mesh,
        out_type=jax.ShapeDtypeStruct((_L, _D), jnp.float32),
        scratch_types=(
            [pltpu.VMEM_SHARED((_NBUF, _CHUNK, _D), jnp.float32)]
            + [pltpu.SemaphoreType.DMA] * (2 * _NBUF)
        ),
    )
    def copy_k(pe_hbm, out_hbm, buf, *sems):
        in_sems = sems[:_NBUF]
        out_sems = sems[_NBUF:]
        cid = lax.axis_index("c")
        base = cid * _ROWS_PER_C

        def in_copy(c):
            b = c % _NBUF
            return pltpu.make_async_copy(
                pe_hbm.at[pl.ds(base + c * _CHUNK, _CHUNK)], buf.at[b], in_sems[b]
            )

        def out_copy(c):
            b = c % _NBUF
            return pltpu.make_async_copy(
                buf.at[b], out_hbm.at[pl.ds(base + c * _CHUNK, _CHUNK)], out_sems[b]
            )

        outs = [None] * _NCHUNK
        ins = [None] * _NCHUNK
        out_waited = [False] * _NCHUNK
        for c in range(min(_NBUF, _NCHUNK)):
            ins[c] = in_copy(c)
            ins[c].start()
        for c in range(_NCHUNK):
            ins[c].wait()
            outs[c] = out_copy(c)
            outs[c].start()
            d = c - _LAG
            if d >= 0 and d + _NBUF < _NCHUNK:
                outs[d].wait()
                out_waited[d] = True
                ins[d + _NBUF] = in_copy(d + _NBUF)
                ins[d + _NBUF].start()
        for c in range(_NCHUNK):
            if not out_waited[c]:
                outs[c].wait()

    return copy_k


_copy_kernel = _make_copy_kernel()


def kernel(x, pe):
    out = _copy_kernel(pe)
    return out[None]


# speedup vs baseline: 1.1173x; 1.1173x over previous
import functools

import jax
import jax.numpy as jnp
from jax import lax
from jax.experimental import pallas as pl
from jax.experimental.pallas import tpu as pltpu
from jax.experimental.pallas import tpu_sc as plsc

_L = 4096
_D = 1024
_NC = 2
_NS = 16
_ROWS_PER_C = _L // _NC
_S_ROWS = 1024
_T_ROWS = _ROWS_PER_C - _S_ROWS
_T_PER_TILE = _T_ROWS // _NS

_S_CHUNK = 256
_S_NCHUNK = _S_ROWS // _S_CHUNK
_S_NBUF = 4

_T_CHUNK = 16
_T_NCHUNK = _T_PER_TILE // _T_CHUNK
_T_NBUF = 4


def _pipeline(src_slab, dst_slab, buf, in_sems, out_sems, nchunk, nbuf, chunk):
    outs = [None] * nchunk
    ins = [None] * nchunk
    out_waited = [False] * nchunk
    for c in range(min(nbuf, nchunk)):
        ins[c] = pltpu.make_async_copy(src_slab(c), buf.at[c % nbuf], in_sems[c % nbuf])
        ins[c].start()
    for c in range(nchunk):
        b = c % nbuf
        ins[c].wait()
        outs[c] = pltpu.make_async_copy(buf.at[b], dst_slab(c), out_sems[b])
        outs[c].start()
        d = c - 2
        if d >= 0 and d + nbuf < nchunk:
            outs[d].wait()
            out_waited[d] = True
            nb = (d + nbuf) % nbuf
            ins[d + nbuf] = pltpu.make_async_copy(
                src_slab(d + nbuf), buf.at[nb], in_sems[nb]
            )
            ins[d + nbuf].start()
    for c in range(nchunk):
        if not out_waited[c]:
            outs[c].wait()


def _make_copy_kernel():
    s_mesh = plsc.ScalarSubcoreMesh(axis_name="c", num_cores=_NC)
    v_mesh = plsc.VectorSubcoreMesh(core_axis_name="c", subcore_axis_name="s")

    def scs_fn(pe_hbm, out_hbm, s_buf, v_buf, *sems):
        del v_buf
        in_sems = sems[:_S_NBUF]
        out_sems = sems[_S_NBUF:2 * _S_NBUF]
        cid = lax.axis_index("c")
        base = cid * _ROWS_PER_C
        _pipeline(
            lambda c: pe_hbm.at[pl.ds(base + c * _S_CHUNK, _S_CHUNK)],
            lambda c: out_hbm.at[0, pl.ds(base + c * _S_CHUNK, _S_CHUNK)],
            s_buf, in_sems, out_sems, _S_NCHUNK, _S_NBUF, _S_CHUNK,
        )

    def tec_fn(pe_hbm, out_hbm, s_buf, v_buf, *sems):
        del s_buf
        in_sems = sems[2 * _S_NBUF:2 * _S_NBUF + _T_NBUF]
        out_sems = sems[2 * _S_NBUF + _T_NBUF:]
        cid = lax.axis_index("c")
        sid = lax.axis_index("s")
        base = cid * _ROWS_PER_C + _S_ROWS + sid * _T_PER_TILE
        _pipeline(
            lambda c: pe_hbm.at[pl.ds(base + c * _T_CHUNK, _T_CHUNK)],
            lambda c: out_hbm.at[0, pl.ds(base + c * _T_CHUNK, _T_CHUNK)],
            v_buf, in_sems, out_sems, _T_NCHUNK, _T_NBUF, _T_CHUNK,
        )

    return pl.kernel(
        body=[scs_fn, tec_fn],
        mesh=[s_mesh, v_mesh],
        out_type=jax.ShapeDtypeStruct((1, _L, _D), jnp.float32),
        scratch_types=(
            [
                pltpu.VMEM_SHARED((_S_NBUF, _S_CHUNK, _D), jnp.float32),
                (pltpu.VMEM @ v_mesh)((_T_NBUF, _T_CHUNK, _D), jnp.float32),
            ]
            + [pltpu.SemaphoreType.DMA @ s_mesh] * (2 * _S_NBUF)
            + [pltpu.SemaphoreType.DMA @ v_mesh] * (2 * _T_NBUF)
        ),
    )


_copy_kernel = _make_copy_kernel()


def kernel(x, pe):
    return _copy_kernel(pe)
